# baseline (speedup 1.0000x reference)
import functools

import jax
import jax.numpy as jnp
from jax import lax
from jax.experimental import pallas as pl
from jax.experimental.pallas import tpu as pltpu
from jax.experimental.pallas import tpu_sc as plsc

_B, _C, _L, _N = 256, 32, 128, 32
_NC, _NS, _LANES = 2, 16, 16
_NW = _NC * _NS
_BPW = _B // _NW
_CH = 8
_NCH = _C // _CH
_STEPS = _BPW * _NCH
_G = _L // _LANES


def _sc_body(x_hbm, a_hbm, b_hbm, out_hbm,
             a_v, b_v, x_v, o_v, sem0, sem1, xsem0, xsem1):
    wid = lax.axis_index("s") * _NC + lax.axis_index("c")
    pltpu.sync_copy(a_hbm, a_v)
    pltpu.sync_copy(b_hbm, b_v)
    sems = (sem0, sem1)
    xsems = (xsem0, xsem1)

    pltpu.async_copy(x_hbm.at[wid * _BPW, pl.ds(0, _CH)], x_v.at[0], xsem0)

    def macro(m, carry):
        for p in range(2):
            i = m * 2 + p
            bb = wid * _BPW + i // _NCH
            c0 = (i % _NCH) * _CH
            obuf = o_v.at[p]
            xbuf = x_v.at[p]
            dst = out_hbm.at[bb, pl.ds(c0, _CH)]

            nxt = jnp.minimum(i + 1, _STEPS - 1)
            bn = wid * _BPW + nxt // _NCH
            c0n = (nxt % _NCH) * _CH

            @pl.when(i < _STEPS - 1)
            def _():
                pltpu.async_copy(x_hbm.at[bn, pl.ds(c0n, _CH)],
                                 x_v.at[1 - p], xsems[1 - p])

            pltpu.make_async_copy(
                x_hbm.at[bb, pl.ds(c0, _CH)], xbuf, xsems[p]).wait()

            @pl.when(m > 0)
            def _():
                pltpu.make_async_copy(obuf, dst, sems[p]).wait()

            zero = jnp.float32(0.0)
            one = jnp.float32(1.0)
            for cp in range(0, _CH, 1):
                cs = (cp,)
                xs = {(c, g): xbuf[c, pl.ds(g * _LANES, _LANES)]
                      for c in cs for g in range(_G)}

                def stage(j, lo_c, hi_c, xs=xs, cs=cs, obuf=obuf, c0=c0):
                    acs = {c: a_v[c0 + c, pl.ds(j, _LANES)][0] for c in cs}
                    bcs = {c: b_v[c0 + c, pl.ds(j, _LANES)][0] for c in cs}
                    rs = {k: xv * acs[k[0]] for k, xv in xs.items()}
                    rs = {k: r + bcs[k[0]] for k, r in rs.items()}
                    if lo_c is not None:
                        rs = {k: jnp.maximum(r, lo_c) for k, r in rs.items()}
                    if hi_c is not None:
                        rs = {k: jnp.minimum(r, hi_c) for k, r in rs.items()}
                    for (c, g), r in rs.items():
                        obuf[c, j, pl.ds(g * _LANES, _LANES)] = r

                stage(0, None, one)
                stage(_N - 1, zero, None)

                @plsc.parallel_loop(1, _N - 1, unroll=2)
                def _(j, stage=stage):
                    stage(j, zero, one)
            pltpu.async_copy(obuf, dst, sems[p])
        return carry

    lax.fori_loop(0, _STEPS // 2, macro, 0, unroll=False)

    last = _STEPS - 1
    for p in range(2):
        i = last - 1 + p
        bb = wid * _BPW + i // _NCH
        c0 = (i % _NCH) * _CH
        pltpu.make_async_copy(
            o_v.at[p], out_hbm.at[bb, pl.ds(c0, _CH)], sems[p]).wait()


def kernel(samples, bin_edges):
    B, C, L = samples.shape
    nb = bin_edges.shape[1] - 1
    e = bin_edges[:, :-1]
    a = 1.0 / (bin_edges[:, 1:] - bin_edges[:, :-1])
    b2 = -e * a
    pad = 48 - nb
    a = jnp.pad(a, ((0, 0), (0, pad)))
    b2 = jnp.pad(b2, ((0, 0), (0, pad)))

    mesh = plsc.VectorSubcoreMesh(core_axis_name="c", subcore_axis_name="s")
    f = pl.kernel(
        _sc_body,
        mesh=mesh,
        out_type=jax.ShapeDtypeStruct((B, C, nb, L), jnp.float32),
        scratch_types=[
            pltpu.VMEM((C, 48), jnp.float32),
            pltpu.VMEM((C, 48), jnp.float32),
            pltpu.VMEM((2, _CH, L), jnp.float32),
            pltpu.VMEM((2, _CH, nb, L), jnp.float32),
            pltpu.SemaphoreType.DMA,
            pltpu.SemaphoreType.DMA,
            pltpu.SemaphoreType.DMA,
            pltpu.SemaphoreType.DMA,
        ],
    )
    out = f(samples, a, b2)
    return out.reshape(B, C * nb, L)


# speedup vs baseline: 1.1044x; 1.1044x over previous
import jax
import jax.numpy as jnp
from jax import lax
from jax.experimental import pallas as pl
from jax.experimental.pallas import tpu as pltpu
from jax.experimental.pallas import tpu_sc as plsc

_B, _C, _L, _N = 256, 32, 128, 32
_NC, _NS, _LANES = 2, 16, 16
_NW = _NC * _NS
_BPW = _B // _NW
_CH = 8
_NCH = _C // _CH
_STEPS = _BPW * _NCH
_G = _L // _LANES


def _sc_body(x_hbm, a_hbm, b_hbm, out_hbm,
             a_v, b_v, x_v, o_v, sem0, sem1, xsem0, xsem1):
    wid = lax.axis_index("s") * _NC + lax.axis_index("c")
    pltpu.sync_copy(a_hbm, a_v)
    pltpu.sync_copy(b_hbm, b_v)
    sems = (sem0, sem1)
    xsems = (xsem0, xsem1)

    pltpu.async_copy(x_hbm.at[wid * _BPW, pl.ds(0, _CH)], x_v.at[0], xsem0)

    def macro(m, carry):
        for p in range(2):
            i = m * 2 + p
            bb = wid * _BPW + i // _NCH
            c0 = (i % _NCH) * _CH
            obuf = o_v.at[p]
            xbuf = x_v.at[p]
            dst = out_hbm.at[bb, pl.ds(c0, _CH)]

            nxt = jnp.minimum(i + 1, _STEPS - 1)
            bn = wid * _BPW + nxt // _NCH
            c0n = (nxt % _NCH) * _CH

            @pl.when(i < _STEPS - 1)
            def _():
                pltpu.async_copy(x_hbm.at[bn, pl.ds(c0n, _CH)],
                                 x_v.at[1 - p], xsems[1 - p])

            pltpu.make_async_copy(
                x_hbm.at[bb, pl.ds(c0, _CH)], xbuf, xsems[p]).wait()

            @pl.when(m > 0)
            def _():
                pltpu.make_async_copy(obuf, dst, sems[p]).wait()

            zero = jnp.float32(0.0)
            one = jnp.float32(1.0)
            for cp in range(0, _CH, 2):
                cs = (cp, cp + 1)
                xs = {(c, g): xbuf[c, pl.ds(g * _LANES, _LANES)]
                      for c in cs for g in range(_G)}

                def stage(j, lo_c, hi_c, xs=xs, cs=cs, obuf=obuf, c0=c0):
                    acs = {c: a_v[c0 + c, pl.ds(j, _LANES)][0] for c in cs}
                    bcs = {c: b_v[c0 + c, pl.ds(j, _LANES)][0] for c in cs}
                    rs = {k: xv * acs[k[0]] for k, xv in xs.items()}
                    rs = {k: r + bcs[k[0]] for k, r in rs.items()}
                    if lo_c is not None:
                        rs = {k: jnp.maximum(r, lo_c) for k, r in rs.items()}
                    if hi_c is not None:
                        rs = {k: jnp.minimum(r, hi_c) for k, r in rs.items()}
                    for (c, g), r in rs.items():
                        obuf[c, j, pl.ds(g * _LANES, _LANES)] = r

                stage(0, None, one)
                stage(_N - 1, zero, None)

                @plsc.parallel_loop(1, _N - 1, unroll=2)
                def _(j, stage=stage):
                    stage(j, zero, one)

                pltpu.async_copy(obuf.at[pl.ds(cp, 2)],
                                 out_hbm.at[bb, pl.ds(c0 + cp, 2)], sems[p])
        return carry

    lax.fori_loop(0, _STEPS // 2, macro, 0, unroll=False)

    last = _STEPS - 1
    for p in range(2):
        i = last - 1 + p
        bb = wid * _BPW + i // _NCH
        c0 = (i % _NCH) * _CH
        pltpu.make_async_copy(
            o_v.at[p], out_hbm.at[bb, pl.ds(c0, _CH)], sems[p]).wait()


def kernel(samples, bin_edges):
    B, C, L = samples.shape
    nb = bin_edges.shape[1] - 1
    e = bin_edges[:, :-1]
    a = 1.0 / (bin_edges[:, 1:] - bin_edges[:, :-1])
    b2 = -e * a
    pad = 48 - nb
    a = jnp.pad(a, ((0, 0), (0, pad)))
    b2 = jnp.pad(b2, ((0, 0), (0, pad)))

    mesh = plsc.VectorSubcoreMesh(core_axis_name="c", subcore_axis_name="s")
    f = pl.kernel(
        _sc_body,
        mesh=mesh,
        out_type=jax.ShapeDtypeStruct((B, C, nb, L), jnp.float32),
        scratch_types=[
            pltpu.VMEM((C, 48), jnp.float32),
            pltpu.VMEM((C, 48), jnp.float32),
            pltpu.VMEM((2, _CH, L), jnp.float32),
            pltpu.VMEM((2, _CH, nb, L), jnp.float32),
            pltpu.SemaphoreType.DMA,
            pltpu.SemaphoreType.DMA,
            pltpu.SemaphoreType.DMA,
            pltpu.SemaphoreType.DMA,
        ],
    )
    out = f(samples, a, b2)
    return out.reshape(B, C * nb, L)
